# baseline (speedup 1.0000x reference)
import numpy as np

import jax
import jax.numpy as jnp
from jax import lax
from jax.experimental import pallas as pl
from jax.experimental.pallas import tpu as pltpu
from jax.experimental.pallas import tpu_sc as plsc

_N = 1600000
_B = 50000
_D = 16
_R8 = 800
_C = 160

_NW = 32
_BPW = 1600
_B_PAD = _NW * _BPW
_GCH = 128
_NG = (_BPW + 1 + _GCH - 1) // _GCH
_G = _NG * _GCH
_PTR_LEN = (_NW - 1) * _BPW + _G


def _lrelu(v):
    return jnp.where(v >= 0, v, 0.2 * v)


def _mlp_cumsum_body(xn, hn, w1xbd, w1hbd, b1t, w2bd, b2t, w3it, cvec,
                     lsc, out, carry):
    @pl.when(pl.program_id(0) == 0)
    def _init():
        carry[...] = jnp.zeros_like(carry)

    x8 = jnp.concatenate([xn[s::8, :] for s in range(8)], axis=1)
    h128 = jnp.concatenate([hn[s::8, :] for s in range(8)], axis=1)
    z1 = _lrelu(x8 @ w1xbd[...] + h128 @ w1hbd[...] + b1t[...])
    z2 = _lrelu(z1 @ w2bd[...] + b2t[...])
    it = z2 @ w3it[...] + cvec[...]

    prev = carry[...]
    for c in range(_R8 // _C):
        tb = it[c * _C:(c + 1) * _C, 128:]
        ic = jax.lax.dot(lsc[...], tb, preferred_element_type=jnp.float32)
        out[c * _C:(c + 1) * _C, :] = (
            it[c * _C:(c + 1) * _C, :128] + ic + prev)
        prev = prev + ic[_C - 1:_C] + tb[_C - 1:_C]
    carry[...] = prev


def _seg_diff_body(e_hbm, ptr_hbm, out_hbm, idx_v, gidx_v, out_v,
                   b0, b1, b2, sem0, sem1, sem2):
    bufs = (b0, b1, b2)
    sems = (sem0, sem1, sem2)
    ptr_s = idx_v
    wid = lax.axis_index("s") * 2 + lax.axis_index("c")
    base = wid * _BPW
    pltpu.sync_copy(ptr_hbm.at[pl.ds(base, _G)], idx_v)

    def mk_gidx(j, _):
        v = idx_v[pl.ds(j * 16, 16)]
        gidx_v[pl.ds(j * 16, 16)] = lax.shift_right_logical(v, 3)
        return 0
    lax.fori_loop(0, _G // 16, mk_gidx, 0)

    def issue(c):
        return pltpu.async_copy(
            e_hbm.at[gidx_v.at[pl.ds(c * _GCH, _GCH)]],
            bufs[c % 3], sems[c % 3])

    pending = [None, None, None]
    pending[0] = issue(0)
    pending[1] = issue(1)
    pending[0].wait()

    def sub16(buf_lo, i_lo, p_lo, buf_hi, i_hi, p_hi):
        lo = buf_lo[i_lo, pl.ds((p_lo & 7) * _D, _D)]
        hi = buf_hi[i_hi, pl.ds((p_hi & 7) * _D, _D)]
        return hi - lo

    for c in range(_NG):
        if c + 2 < _NG:
            pending[(c + 2) % 3] = issue(c + 2)
        if c + 1 < _NG:
            pending[(c + 1) % 3].wait()
        ba = bufs[c % 3]
        nrows = min(_GCH, _BPW - c * _GCH)
        in_chunk = nrows if nrows < _GCH else _GCH - 1

        def row(i, _):
            s = c * _GCH + i
            pv = ptr_s[pl.ds(s, 16)]
            p_lo = pv[0]
            p_hi = pv[1]
            out_v[s >> 3, pl.ds((s & 7) * _D, _D)] = sub16(
                ba, i, p_lo, ba, i + 1, p_hi)
            return 0
        lax.fori_loop(0, in_chunk, row, 0)
        if in_chunk < nrows:
            s = c * _GCH + _GCH - 1
            pv = ptr_s[pl.ds(s, 16)]
            p_lo = pv[0]
            p_hi = pv[1]
            out_v[s >> 3, pl.ds((s & 7) * _D, _D)] = sub16(
                ba, _GCH - 1, p_lo, bufs[(c + 1) % 3], 0, p_hi)

    pltpu.sync_copy(out_v, out_hbm.at[pl.ds(wid * (_BPW // 8), _BPW // 8)])


def _block_diag8(w):
    k, m = w.shape
    out = jnp.zeros((8 * k, 8 * m), dtype=w.dtype)
    for s in range(8):
        out = out.at[s * k:(s + 1) * k, s * m:(s + 1) * m].set(w)
    return out


def kernel(h_node, x, ptr, W1, b1, W2, b2, W3, b3):
    n = h_node.shape[0]
    nf = x.shape[1]
    n8 = n // 8

    w1xbd = _block_diag8(W1[:, :nf].T)
    w1hbd = _block_diag8(W1[:, nf:].T)
    w2bd = _block_diag8(W2.T)
    w3bd = _block_diag8(W3.T)
    b1t = jnp.tile(b1, 8).reshape(1, -1)
    b2t = jnp.tile(b2, 8).reshape(1, -1)
    b3t = jnp.tile(b3, 8).reshape(1, -1)

    l_intra = jnp.asarray(np.kron(np.triu(np.ones((8, 8)), 1),
                                  np.eye(_D)), dtype=jnp.float32)
    t_bcast = jnp.asarray(np.kron(np.ones((8, 8)), np.eye(_D)),
                          dtype=jnp.float32)
    w3it = jnp.concatenate([w3bd @ l_intra, w3bd @ t_bcast], axis=1)
    cvec = jnp.concatenate([b3t @ l_intra, b3t @ t_bcast], axis=1)
    lsc = jnp.asarray(np.tril(np.ones((_C, _C)), -1), dtype=jnp.float32)

    e2 = pl.pallas_call(
        _mlp_cumsum_body,
        grid=(n8 // _R8,),
        in_specs=[
            pl.BlockSpec((8 * _R8, nf), lambda i: (i, 0)),
            pl.BlockSpec((8 * _R8, _D), lambda i: (i, 0)),
            pl.BlockSpec(w1xbd.shape, lambda i: (0, 0)),
            pl.BlockSpec(w1hbd.shape, lambda i: (0, 0)),
            pl.BlockSpec(b1t.shape, lambda i: (0, 0)),
            pl.BlockSpec(w2bd.shape, lambda i: (0, 0)),
            pl.BlockSpec(b2t.shape, lambda i: (0, 0)),
            pl.BlockSpec(w3it.shape, lambda i: (0, 0)),
            pl.BlockSpec(cvec.shape, lambda i: (0, 0)),
            pl.BlockSpec(lsc.shape, lambda i: (0, 0)),
        ],
        out_specs=pl.BlockSpec((_R8, 8 * _D), lambda i: (i, 0)),
        out_shape=jax.ShapeDtypeStruct((n8, 8 * _D), jnp.float32),
        scratch_shapes=[pltpu.VMEM((1, 8 * _D), jnp.float32)],
    )(x, h_node, w1xbd, w1hbd, b1t, w2bd, b2t, w3it, cvec, lsc)

    ptr32 = jnp.pad(ptr.astype(jnp.int32), (0, _PTR_LEN - (_B + 1)),
                    mode="edge")

    seg = pl.kernel(
        _seg_diff_body,
        out_type=jax.ShapeDtypeStruct((_B_PAD // 8, 8 * _D), jnp.float32),
        mesh=plsc.VectorSubcoreMesh(core_axis_name="c", subcore_axis_name="s"),
        scratch_types=[
            pltpu.VMEM((_G,), jnp.int32),
            pltpu.VMEM((_G,), jnp.int32),
            pltpu.VMEM((_BPW // 8, 8 * _D), jnp.float32),
            pltpu.VMEM((_GCH, 8 * _D), jnp.float32),
            pltpu.VMEM((_GCH, 8 * _D), jnp.float32),
            pltpu.VMEM((_GCH, 8 * _D), jnp.float32),
            pltpu.SemaphoreType.DMA,
            pltpu.SemaphoreType.DMA,
            pltpu.SemaphoreType.DMA,
        ],
    )(e2, ptr32)

    return seg.reshape(_B_PAD, _D)[:_B]


# speedup vs baseline: 2.3456x; 2.3456x over previous
import numpy as np

import jax
import jax.numpy as jnp
from jax import lax
from jax.experimental import pallas as pl
from jax.experimental.pallas import tpu as pltpu
from jax.experimental.pallas import tpu_sc as plsc

_N = 1600000
_B = 50000
_D = 16
_GL = 1024
_NBLK = 196
_SL = _GL * _NBLK
_CL = 256

_NW = 32
_BPW = 1600
_B_PAD = _NW * _BPW
_GCH = 128
_NG = (_BPW + 1 + _GCH - 1) // _GCH
_G = _NG * _GCH
_PTR_LEN = (_NW - 1) * _BPW + _G


def _lrelu(v):
    return jnp.where(v >= 0, v, 0.2 * v)


def _mlp_cumsum_body(*refs):
    xts = refs[0:8]
    hts = refs[8:16]
    w1big, b1big, w2big, b2big, w3big, b3big, us = refs[16:23]
    out, tot_out, carry = refs[23:26]

    @pl.when(pl.program_id(0) == 0)
    def _init():
        carry[...] = jnp.zeros_like(carry)

    parts = []
    zpad = jnp.zeros((3, _GL), jnp.float32)
    for s in range(8):
        parts += [xts[s][...], zpad, hts[s][...]]
    ins = jnp.concatenate(parts, axis=0)

    z1 = _lrelu(w1big[...] @ ins + b1big[...])
    z2 = _lrelu(w2big[...] @ z1 + b2big[...])
    h3 = w3big[...] @ z2 + b3big[...]

    i = pl.program_id(0)
    thr = ((_N - i * _GL)
           - (lax.broadcasted_iota(jnp.int32, (128, 1), 0) >> 4) * _SL)
    lane = lax.broadcasted_iota(jnp.int32, (1, _GL), 1)
    h3 = jnp.where(lane < thr, h3, 0.0)

    chunks = []
    for c in range(_GL // _CL):
        hc = h3[:, c * _CL:(c + 1) * _CL]
        et = jax.lax.dot(hc, us[...],
                         preferred_element_type=jnp.float32) + carry[...]
        chunks.append(et)
        tot = et[:, _CL - 1:_CL] + hc[:, _CL - 1:_CL]
        carry[...] = tot
    etb = jnp.concatenate(chunks, axis=1)
    out[...] = jnp.swapaxes(etb, 0, 1)
    tot_out[...] = carry[...]


def _seg_diff_body(e_hbm, ptr_hbm, out_hbm, idx_v, gidx_v, olane_v, out_v,
                   b0, b1, b2, sem0, sem1, sem2):
    bufs = (b0, b1, b2)
    sems = (sem0, sem1, sem2)
    wid = lax.axis_index("s") * 2 + lax.axis_index("c")
    base = wid * _BPW
    pltpu.sync_copy(ptr_hbm.at[pl.ds(base, _G)], idx_v)

    def mk_gidx(j, _):
        v = idx_v[pl.ds(j * 16, 16)]
        s16 = jnp.zeros((16,), jnp.int32)
        for t in range(1, 8):
            s16 = s16 + jnp.where(v >= t * _SL, 1, 0).astype(jnp.int32)
        gidx_v[pl.ds(j * 16, 16)] = v - s16 * _SL
        olane_v[pl.ds(j * 16, 16)] = s16 * _D
        return 0
    lax.fori_loop(0, _G // 16, mk_gidx, 0)

    def issue(c):
        return pltpu.async_copy(
            e_hbm.at[gidx_v.at[pl.ds(c * _GCH, _GCH)]],
            bufs[c % 3], sems[c % 3])

    pending = [None, None, None]
    pending[0] = issue(0)
    pending[1] = issue(1)
    pending[0].wait()

    def sub16(buf_lo, i_lo, o_lo, buf_hi, i_hi, o_hi):
        lo = buf_lo[i_lo, pl.ds(o_lo, _D)]
        hi = buf_hi[i_hi, pl.ds(o_hi, _D)]
        return hi - lo

    for c in range(_NG):
        if c + 2 < _NG:
            pending[(c + 2) % 3] = issue(c + 2)
        if c + 1 < _NG:
            pending[(c + 1) % 3].wait()
        ba = bufs[c % 3]
        nrows = min(_GCH, _BPW - c * _GCH)
        in_chunk = nrows if nrows < _GCH else _GCH - 1

        def row(i, _):
            s = c * _GCH + i
            ov = olane_v[pl.ds(s, 16)]
            out_v[s >> 3, pl.ds((s & 7) * _D, _D)] = sub16(
                ba, i, ov[0], ba, i + 1, ov[1])
            return 0
        lax.fori_loop(0, in_chunk, row, 0)
        if in_chunk < nrows:
            s = c * _GCH + _GCH - 1
            ov = olane_v[pl.ds(s, 16)]
            out_v[s >> 3, pl.ds((s & 7) * _D, _D)] = sub16(
                ba, _GCH - 1, ov[0], bufs[(c + 1) % 3], 0, ov[1])

    pltpu.sync_copy(out_v, out_hbm.at[pl.ds(wid * (_BPW // 8), _BPW // 8)])


def kernel(h_node, x, ptr, W1, b1, W2, b2, W3, b3):
    n = h_node.shape[0]
    nf = x.shape[1]
    n8 = n // 8

    xt = x.T
    ht = h_node.T

    w1x = W1[:, :nf]
    w1h = W1[:, nf:]
    w1big = jnp.zeros((256, 192), jnp.float32)
    w2big = jnp.zeros((128, 256), jnp.float32)
    w3big = jnp.zeros((128, 128), jnp.float32)
    for s in range(8):
        w1big = w1big.at[32 * s:32 * s + 32, 24 * s:24 * s + 5].set(w1x)
        w1big = w1big.at[32 * s:32 * s + 32, 24 * s + 8:24 * s + 24].set(w1h)
        w2big = w2big.at[16 * s:16 * s + 16, 32 * s:32 * s + 32].set(W2)
        w3big = w3big.at[16 * s:16 * s + 16, 16 * s:16 * s + 16].set(W3)
    b1big = jnp.tile(b1, 8).reshape(-1, 1)
    b2big = jnp.tile(b2, 8).reshape(-1, 1)
    b3big = jnp.tile(b3, 8).reshape(-1, 1)
    us = jnp.asarray(np.triu(np.ones((_CL, _CL)), 1), dtype=jnp.float32)

    nlastblk = (n + _GL - 1) // _GL - 1
    xspec = [pl.BlockSpec(
                 (nf, _GL),
                 (lambda s: (lambda i: (0, jnp.minimum(s * _NBLK + i,
                                                       nlastblk))))(s))
             for s in range(8)]
    hspec = [pl.BlockSpec(
                 (_D, _GL),
                 (lambda s: (lambda i: (0, jnp.minimum(s * _NBLK + i,
                                                       nlastblk))))(s))
             for s in range(8)]

    e2, tots = pl.pallas_call(
        _mlp_cumsum_body,
        grid=(_NBLK,),
        in_specs=xspec + hspec + [
            pl.BlockSpec(w1big.shape, lambda i: (0, 0)),
            pl.BlockSpec(b1big.shape, lambda i: (0, 0)),
            pl.BlockSpec(w2big.shape, lambda i: (0, 0)),
            pl.BlockSpec(b2big.shape, lambda i: (0, 0)),
            pl.BlockSpec(w3big.shape, lambda i: (0, 0)),
            pl.BlockSpec(b3big.shape, lambda i: (0, 0)),
            pl.BlockSpec(us.shape, lambda i: (0, 0)),
        ],
        out_specs=[pl.BlockSpec((_GL, 8 * _D), lambda i: (i, 0)),
                   pl.BlockSpec((128, 1), lambda i: (0, 0))],
        out_shape=[jax.ShapeDtypeStruct((_SL, 8 * _D), jnp.float32),
                   jax.ShapeDtypeStruct((128, 1), jnp.float32)],
        scratch_shapes=[pltpu.VMEM((128, 1), jnp.float32)],
    )(*([xt] * 8 + [ht] * 8 + [w1big, b1big, w2big, b2big, w3big, b3big, us]))

    ptr32 = jnp.pad(ptr.astype(jnp.int32), (0, _PTR_LEN - (_B + 1)),
                    mode="edge")

    seg = pl.kernel(
        _seg_diff_body,
        out_type=jax.ShapeDtypeStruct((_B_PAD // 8, 8 * _D), jnp.float32),
        mesh=plsc.VectorSubcoreMesh(core_axis_name="c", subcore_axis_name="s"),
        scratch_types=[
            pltpu.VMEM((_G,), jnp.int32),
            pltpu.VMEM((_G,), jnp.int32),
            pltpu.VMEM((_G,), jnp.int32),
            pltpu.VMEM((_BPW // 8, 8 * _D), jnp.float32),
            pltpu.VMEM((_GCH, 8 * _D), jnp.float32),
            pltpu.VMEM((_GCH, 8 * _D), jnp.float32),
            pltpu.VMEM((_GCH, 8 * _D), jnp.float32),
            pltpu.SemaphoreType.DMA,
            pltpu.SemaphoreType.DMA,
            pltpu.SemaphoreType.DMA,
        ],
    )(e2, ptr32)

    t8 = tots.reshape(8, _D)
    off = jnp.cumsum(t8, axis=0) - t8
    p32 = ptr.astype(jnp.int32)
    s_lo = p32[:-1] // _SL
    s_hi = p32[1:] // _SL
    out = seg.reshape(_B_PAD, _D)[:_B] + off[s_hi] - off[s_lo]
    return out


# speedup vs baseline: 2.9445x; 1.2553x over previous
import numpy as np

import jax
import jax.numpy as jnp
from jax import lax
from jax.experimental import pallas as pl
from jax.experimental.pallas import tpu as pltpu
from jax.experimental.pallas import tpu_sc as plsc

_N = 1600000
_B = 50000
_D = 16
_GL = 1024
_NBLK = 196
_SL = _GL * _NBLK
_CL = 256

_NW = 32
_BPW = 1600
_B_PAD = _NW * _BPW
_GCH = 128
_NG = (_BPW + 1 + _GCH - 1) // _GCH
_G = _NG * _GCH
_PTR_LEN = (_NW - 1) * _BPW + _G


def _lrelu(v):
    return jnp.where(v >= 0, v, 0.2 * v)


def _mlp_cumsum_body(*refs):
    xts = refs[0:8]
    hts = refs[8:16]
    w1big, b1big, w2big, b2big, w3big, b3big, us = refs[16:23]
    out, tot_out, carry = refs[23:26]

    @pl.when(pl.program_id(0) == 0)
    def _init():
        carry[...] = jnp.zeros_like(carry)

    parts = []
    zpad = jnp.zeros((3, _GL), jnp.float32)
    for s in range(8):
        parts += [xts[s][...], zpad, hts[s][...]]
    ins = jnp.concatenate(parts, axis=0)

    z1 = _lrelu(w1big[...] @ ins + b1big[...])
    z2 = _lrelu(w2big[...] @ z1 + b2big[...])
    h3 = w3big[...] @ z2 + b3big[...]

    i = pl.program_id(0)
    thr = ((_N - i * _GL)
           - (lax.broadcasted_iota(jnp.int32, (128, 1), 0) >> 4) * _SL)
    lane = lax.broadcasted_iota(jnp.int32, (1, _GL), 1)
    h3 = jnp.where(lane < thr, h3, 0.0)

    chunks = []
    for c in range(_GL // _CL):
        hc = h3[:, c * _CL:(c + 1) * _CL]
        et = jax.lax.dot(hc, us[...],
                         preferred_element_type=jnp.float32) + carry[...]
        chunks.append(et)
        tot = et[:, _CL - 1:_CL] + hc[:, _CL - 1:_CL]
        carry[...] = tot
    etb = jnp.concatenate(chunks, axis=1)
    out[...] = jnp.swapaxes(etb, 0, 1)
    tot_out[...] = carry[...]


def _seg_diff_body(e_hbm, ptr_hbm, off_hbm, out_hbm, idx_v, gidx_v, olane_v,
                   off_v, out_v, b0, b1, b2, sem0, sem1, sem2):
    bufs = (b0, b1, b2)
    sems = (sem0, sem1, sem2)
    wid = lax.axis_index("s") * 2 + lax.axis_index("c")
    base = wid * _BPW
    pltpu.sync_copy(off_hbm, off_v)
    pltpu.sync_copy(ptr_hbm.at[pl.ds(base, _G)], idx_v)

    def mk_gidx(j, _):
        v = idx_v[pl.ds(j * 16, 16)]
        s16 = jnp.zeros((16,), jnp.int32)
        for t in range(1, 8):
            s16 = s16 + jnp.where(v >= t * _SL, 1, 0).astype(jnp.int32)
        gidx_v[pl.ds(j * 16, 16)] = v - s16 * _SL
        olane_v[pl.ds(j * 16, 16)] = s16 * _D
        return 0
    lax.fori_loop(0, _G // 16, mk_gidx, 0)

    def issue(c):
        return pltpu.async_copy(
            e_hbm.at[gidx_v.at[pl.ds(c * _GCH, _GCH)]],
            bufs[c % 3], sems[c % 3])

    pending = [None, None, None]
    pending[0] = issue(0)
    pending[1] = issue(1)
    pending[0].wait()

    def sub16(buf_lo, i_lo, o_lo, buf_hi, i_hi, o_hi):
        lo = buf_lo[i_lo, pl.ds(o_lo, _D)] + off_v[pl.ds(o_lo, _D)]
        hi = buf_hi[i_hi, pl.ds(o_hi, _D)] + off_v[pl.ds(o_hi, _D)]
        return hi - lo

    for c in range(_NG):
        if c + 2 < _NG:
            pending[(c + 2) % 3] = issue(c + 2)
        if c + 1 < _NG:
            pending[(c + 1) % 3].wait()
        ba = bufs[c % 3]
        nrows = min(_GCH, _BPW - c * _GCH)
        in_chunk = nrows if nrows < _GCH else _GCH - 1

        def row(i, _):
            s = c * _GCH + i
            ov = olane_v[pl.ds(s, 16)]
            out_v[s >> 3, pl.ds((s & 7) * _D, _D)] = sub16(
                ba, i, ov[0], ba, i + 1, ov[1])
            return 0
        lax.fori_loop(0, in_chunk, row, 0)
        if in_chunk < nrows:
            s = c * _GCH + _GCH - 1
            ov = olane_v[pl.ds(s, 16)]
            out_v[s >> 3, pl.ds((s & 7) * _D, _D)] = sub16(
                ba, _GCH - 1, ov[0], bufs[(c + 1) % 3], 0, ov[1])

    pltpu.sync_copy(out_v, out_hbm.at[pl.ds(wid * (_BPW // 8), _BPW // 8)])


def kernel(h_node, x, ptr, W1, b1, W2, b2, W3, b3):
    n = h_node.shape[0]
    nf = x.shape[1]
    n8 = n // 8

    xt = x.T
    ht = h_node.T

    w1x = W1[:, :nf]
    w1h = W1[:, nf:]
    w1big = jnp.zeros((256, 192), jnp.float32)
    w2big = jnp.zeros((128, 256), jnp.float32)
    w3big = jnp.zeros((128, 128), jnp.float32)
    for s in range(8):
        w1big = w1big.at[32 * s:32 * s + 32, 24 * s:24 * s + 5].set(w1x)
        w1big = w1big.at[32 * s:32 * s + 32, 24 * s + 8:24 * s + 24].set(w1h)
        w2big = w2big.at[16 * s:16 * s + 16, 32 * s:32 * s + 32].set(W2)
        w3big = w3big.at[16 * s:16 * s + 16, 16 * s:16 * s + 16].set(W3)
    b1big = jnp.tile(b1, 8).reshape(-1, 1)
    b2big = jnp.tile(b2, 8).reshape(-1, 1)
    b3big = jnp.tile(b3, 8).reshape(-1, 1)
    us = jnp.asarray(np.triu(np.ones((_CL, _CL)), 1), dtype=jnp.float32)

    nlastblk = (n + _GL - 1) // _GL - 1
    xspec = [pl.BlockSpec(
                 (nf, _GL),
                 (lambda s: (lambda i: (0, jnp.minimum(s * _NBLK + i,
                                                       nlastblk))))(s))
             for s in range(8)]
    hspec = [pl.BlockSpec(
                 (_D, _GL),
                 (lambda s: (lambda i: (0, jnp.minimum(s * _NBLK + i,
                                                       nlastblk))))(s))
             for s in range(8)]

    e2, tots = pl.pallas_call(
        _mlp_cumsum_body,
        grid=(_NBLK,),
        in_specs=xspec + hspec + [
            pl.BlockSpec(w1big.shape, lambda i: (0, 0)),
            pl.BlockSpec(b1big.shape, lambda i: (0, 0)),
            pl.BlockSpec(w2big.shape, lambda i: (0, 0)),
            pl.BlockSpec(b2big.shape, lambda i: (0, 0)),
            pl.BlockSpec(w3big.shape, lambda i: (0, 0)),
            pl.BlockSpec(b3big.shape, lambda i: (0, 0)),
            pl.BlockSpec(us.shape, lambda i: (0, 0)),
        ],
        out_specs=[pl.BlockSpec((_GL, 8 * _D), lambda i: (i, 0)),
                   pl.BlockSpec((128, 1), lambda i: (0, 0))],
        out_shape=[jax.ShapeDtypeStruct((_SL, 8 * _D), jnp.float32),
                   jax.ShapeDtypeStruct((128, 1), jnp.float32)],
        scratch_shapes=[pltpu.VMEM((128, 1), jnp.float32)],
    )(*([xt] * 8 + [ht] * 8 + [w1big, b1big, w2big, b2big, w3big, b3big, us]))

    ptr32 = jnp.pad(ptr.astype(jnp.int32), (0, _PTR_LEN - (_B + 1)),
                    mode="edge")

    t8 = tots.reshape(8, _D)
    off_flat = (jnp.cumsum(t8, axis=0) - t8).reshape(8 * _D)

    seg = pl.kernel(
        _seg_diff_body,
        out_type=jax.ShapeDtypeStruct((_B_PAD // 8, 8 * _D), jnp.float32),
        mesh=plsc.VectorSubcoreMesh(core_axis_name="c", subcore_axis_name="s"),
        scratch_types=[
            pltpu.VMEM((_G,), jnp.int32),
            pltpu.VMEM((_G,), jnp.int32),
            pltpu.VMEM((_G,), jnp.int32),
            pltpu.VMEM((8 * _D,), jnp.float32),
            pltpu.VMEM((_BPW // 8, 8 * _D), jnp.float32),
            pltpu.VMEM((_GCH, 8 * _D), jnp.float32),
            pltpu.VMEM((_GCH, 8 * _D), jnp.float32),
            pltpu.VMEM((_GCH, 8 * _D), jnp.float32),
            pltpu.SemaphoreType.DMA,
            pltpu.SemaphoreType.DMA,
            pltpu.SemaphoreType.DMA,
        ],
    )(e2, ptr32, off_flat)

    return seg.reshape(_B_PAD, _D)[:_B]


# speedup vs baseline: 3.0966x; 1.0516x over previous
import numpy as np

import jax
import jax.numpy as jnp
from jax import lax
from jax.experimental import pallas as pl
from jax.experimental.pallas import tpu as pltpu
from jax.experimental.pallas import tpu_sc as plsc

_N = 1600000
_B = 50000
_D = 16
_GL = 2048
_NBLK = 98
_SL = _GL * _NBLK
_CL = 256

_NW = 32
_BPW = 1600
_B_PAD = _NW * _BPW
_GCH = 128
_NG = (_BPW + 1 + _GCH - 1) // _GCH
_G = _NG * _GCH
_PTR_LEN = (_NW - 1) * _BPW + _G


def _lrelu(v):
    return jnp.where(v >= 0, v, 0.2 * v)


def _mlp_cumsum_body(*refs):
    xts = refs[0:8]
    hts = refs[8:16]
    w1big, b1big, w2big, b2big, w3big, b3big, us = refs[16:23]
    out, tot_out, carry = refs[23:26]

    @pl.when(pl.program_id(0) == 0)
    def _init():
        carry[...] = jnp.zeros_like(carry)

    parts = []
    zpad = jnp.zeros((3, _GL), jnp.float32)
    for s in range(8):
        parts += [xts[s][...], zpad, hts[s][...]]
    ins = jnp.concatenate(parts, axis=0)

    i = pl.program_id(0)
    thr = ((_N - i * _GL)
           - (lax.broadcasted_iota(jnp.int32, (128, 1), 0) >> 4) * _SL)
    lane = lax.broadcasted_iota(jnp.int32, (1, _CL), 1)

    hcs = []
    ets = []
    for c in range(_GL // _CL):
        insc = ins[:, c * _CL:(c + 1) * _CL]
        z1 = _lrelu(w1big[...] @ insc + b1big[...])
        z2 = _lrelu(w2big[...] @ z1 + b2big[...])
        h3 = w3big[...] @ z2 + b3big[...]
        h3 = jnp.where(lane + c * _CL < thr, h3, 0.0)
        hcs.append(h3)
        ets.append(jax.lax.dot(h3, us[...],
                               preferred_element_type=jnp.float32))

    for c in range(_GL // _CL):
        et = ets[c] + carry[...]
        out[c * _CL:(c + 1) * _CL, :] = jnp.swapaxes(et, 0, 1)
        carry[...] = et[:, _CL - 1:_CL] + hcs[c][:, _CL - 1:_CL]
    tot_out[...] = carry[...]


def _seg_diff_body(e_hbm, ptr_hbm, off_hbm, out_hbm, idx_v, gidx_v, olane_v,
                   off_v, out_v, b0, b1, b2, sem0, sem1, sem2):
    bufs = (b0, b1, b2)
    sems = (sem0, sem1, sem2)
    wid = lax.axis_index("s") * 2 + lax.axis_index("c")
    base = wid * _BPW
    pltpu.sync_copy(off_hbm, off_v)
    pltpu.sync_copy(ptr_hbm.at[pl.ds(base, _G)], idx_v)

    def mk_gidx(j, _):
        v = idx_v[pl.ds(j * 16, 16)]
        s16 = jnp.zeros((16,), jnp.int32)
        for t in range(1, 8):
            s16 = s16 + jnp.where(v >= t * _SL, 1, 0).astype(jnp.int32)
        gidx_v[pl.ds(j * 16, 16)] = v - s16 * _SL
        olane_v[pl.ds(j * 16, 16)] = s16 * _D
        return 0
    lax.fori_loop(0, _G // 16, mk_gidx, 0)

    def issue(c):
        return pltpu.async_copy(
            e_hbm.at[gidx_v.at[pl.ds(c * _GCH, _GCH)]],
            bufs[c % 3], sems[c % 3])

    pending = [None, None, None]
    pending[0] = issue(0)
    pending[1] = issue(1)
    pending[0].wait()

    def sub16(buf_lo, i_lo, o_lo, buf_hi, i_hi, o_hi):
        lo = buf_lo[i_lo, pl.ds(o_lo, _D)] + off_v[pl.ds(o_lo, _D)]
        hi = buf_hi[i_hi, pl.ds(o_hi, _D)] + off_v[pl.ds(o_hi, _D)]
        return hi - lo

    for c in range(_NG):
        if c + 2 < _NG:
            pending[(c + 2) % 3] = issue(c + 2)
        if c + 1 < _NG:
            pending[(c + 1) % 3].wait()
        ba = bufs[c % 3]
        nrows = min(_GCH, _BPW - c * _GCH)
        in_chunk = nrows if nrows < _GCH else _GCH - 1

        def row(i, _):
            s = c * _GCH + i
            ov = olane_v[pl.ds(s, 16)]
            out_v[s >> 3, pl.ds((s & 7) * _D, _D)] = sub16(
                ba, i, ov[0], ba, i + 1, ov[1])
            return 0
        lax.fori_loop(0, in_chunk, row, 0)
        if in_chunk < nrows:
            s = c * _GCH + _GCH - 1
            ov = olane_v[pl.ds(s, 16)]
            out_v[s >> 3, pl.ds((s & 7) * _D, _D)] = sub16(
                ba, _GCH - 1, ov[0], bufs[(c + 1) % 3], 0, ov[1])

    pltpu.sync_copy(out_v, out_hbm.at[pl.ds(wid * (_BPW // 8), _BPW // 8)])


def kernel(h_node, x, ptr, W1, b1, W2, b2, W3, b3):
    n = h_node.shape[0]
    nf = x.shape[1]
    n8 = n // 8

    xt = x.T
    ht = h_node.T

    w1x = W1[:, :nf]
    w1h = W1[:, nf:]
    w1big = jnp.zeros((256, 192), jnp.float32)
    w2big = jnp.zeros((128, 256), jnp.float32)
    w3big = jnp.zeros((128, 128), jnp.float32)
    for s in range(8):
        w1big = w1big.at[32 * s:32 * s + 32, 24 * s:24 * s + 5].set(w1x)
        w1big = w1big.at[32 * s:32 * s + 32, 24 * s + 8:24 * s + 24].set(w1h)
        w2big = w2big.at[16 * s:16 * s + 16, 32 * s:32 * s + 32].set(W2)
        w3big = w3big.at[16 * s:16 * s + 16, 16 * s:16 * s + 16].set(W3)
    b1big = jnp.tile(b1, 8).reshape(-1, 1)
    b2big = jnp.tile(b2, 8).reshape(-1, 1)
    b3big = jnp.tile(b3, 8).reshape(-1, 1)
    us = jnp.asarray(np.triu(np.ones((_CL, _CL)), 1), dtype=jnp.float32)

    nlastblk = (n + _GL - 1) // _GL - 1

    def _mkmap(s):
        if (s + 1) * _SL <= n:
            return lambda i: (0, s * _NBLK + i)
        return lambda i: (0, jnp.minimum(s * _NBLK + i, nlastblk))

    xspec = [pl.BlockSpec((nf, _GL), _mkmap(s)) for s in range(8)]
    hspec = [pl.BlockSpec((_D, _GL), _mkmap(s)) for s in range(8)]

    e2, tots = pl.pallas_call(
        _mlp_cumsum_body,
        grid=(_NBLK,),
        in_specs=xspec + hspec + [
            pl.BlockSpec(w1big.shape, lambda i: (0, 0)),
            pl.BlockSpec(b1big.shape, lambda i: (0, 0)),
            pl.BlockSpec(w2big.shape, lambda i: (0, 0)),
            pl.BlockSpec(b2big.shape, lambda i: (0, 0)),
            pl.BlockSpec(w3big.shape, lambda i: (0, 0)),
            pl.BlockSpec(b3big.shape, lambda i: (0, 0)),
            pl.BlockSpec(us.shape, lambda i: (0, 0)),
        ],
        out_specs=[pl.BlockSpec((_GL, 8 * _D), lambda i: (i, 0)),
                   pl.BlockSpec((128, 1), lambda i: (0, 0))],
        out_shape=[jax.ShapeDtypeStruct((_SL, 8 * _D), jnp.float32),
                   jax.ShapeDtypeStruct((128, 1), jnp.float32)],
        scratch_shapes=[pltpu.VMEM((128, 1), jnp.float32)],
    )(*([xt] * 8 + [ht] * 8 + [w1big, b1big, w2big, b2big, w3big, b3big, us]))

    ptr32 = jnp.pad(ptr.astype(jnp.int32), (0, _PTR_LEN - (_B + 1)),
                    mode="edge")

    t8 = tots.reshape(8, _D)
    off_flat = (jnp.cumsum(t8, axis=0) - t8).reshape(8 * _D)

    seg = pl.kernel(
        _seg_diff_body,
        out_type=jax.ShapeDtypeStruct((_B_PAD // 8, 8 * _D), jnp.float32),
        mesh=plsc.VectorSubcoreMesh(core_axis_name="c", subcore_axis_name="s"),
        scratch_types=[
            pltpu.VMEM((_G,), jnp.int32),
            pltpu.VMEM((_G,), jnp.int32),
            pltpu.VMEM((_G,), jnp.int32),
            pltpu.VMEM((8 * _D,), jnp.float32),
            pltpu.VMEM((_BPW // 8, 8 * _D), jnp.float32),
            pltpu.VMEM((_GCH, 8 * _D), jnp.float32),
            pltpu.VMEM((_GCH, 8 * _D), jnp.float32),
            pltpu.VMEM((_GCH, 8 * _D), jnp.float32),
            pltpu.SemaphoreType.DMA,
            pltpu.SemaphoreType.DMA,
            pltpu.SemaphoreType.DMA,
        ],
    )(e2, ptr32, off_flat)

    return seg.reshape(_B_PAD, _D)[:_B]


# speedup vs baseline: 3.2597x; 1.0527x over previous
import numpy as np

import jax
import jax.numpy as jnp
from jax import lax
from jax.experimental import pallas as pl
from jax.experimental.pallas import tpu as pltpu
from jax.experimental.pallas import tpu_sc as plsc

_N = 1600000
_B = 50000
_D = 16
_GL = 4096
_NBLK = 49
_SL = _GL * _NBLK
_CL = 256

_NW = 32
_BPW = 1600
_B_PAD = _NW * _BPW
_GCH = 128
_NG = (_BPW + 1 + _GCH - 1) // _GCH
_G = _NG * _GCH
_PTR_LEN = (_NW - 1) * _BPW + _G


def _lrelu(v):
    return jnp.where(v >= 0, v, 0.2 * v)


def _mlp_cumsum_body(*refs):
    xts = refs[0:8]
    hts = refs[8:16]
    w1big, b1big, w2big, b2big, w3big, b3big, us = refs[16:23]
    out, tot_out, carry = refs[23:26]

    @pl.when(pl.program_id(0) == 0)
    def _init():
        carry[...] = jnp.zeros_like(carry)

    parts = []
    zpad = jnp.zeros((3, _GL), jnp.float32)
    for s in range(8):
        parts += [xts[s][...], zpad, hts[s][...]]
    ins = jnp.concatenate(parts, axis=0)

    i = pl.program_id(0)
    thr = ((_N - i * _GL)
           - (lax.broadcasted_iota(jnp.int32, (128, 1), 0) >> 4) * _SL)
    lane = lax.broadcasted_iota(jnp.int32, (1, _CL), 1)

    hcs = []
    ets = []
    for c in range(_GL // _CL):
        insc = ins[:, c * _CL:(c + 1) * _CL]
        z1 = _lrelu(w1big[...] @ insc + b1big[...])
        z2 = _lrelu(w2big[...] @ z1 + b2big[...])
        h3 = w3big[...] @ z2 + b3big[...]
        h3 = jnp.where(lane + c * _CL < thr, h3, 0.0)
        hcs.append(h3)
        ets.append(jax.lax.dot(h3, us[...],
                               preferred_element_type=jnp.float32))

    for c in range(_GL // _CL):
        et = ets[c] + carry[...]
        out[c * _CL:(c + 1) * _CL, :] = jnp.swapaxes(et, 0, 1)
        carry[...] = et[:, _CL - 1:_CL] + hcs[c][:, _CL - 1:_CL]
    tot_out[...] = carry[...]


def _seg_diff_body(e_hbm, ptr_hbm, off_hbm, out_hbm, idx_v, gidx_v, olane_v,
                   off_v, out_v, b0, b1, b2, sem0, sem1, sem2):
    bufs = (b0, b1, b2)
    sems = (sem0, sem1, sem2)
    wid = lax.axis_index("s") * 2 + lax.axis_index("c")
    base = wid * _BPW
    pltpu.sync_copy(off_hbm, off_v)
    pltpu.sync_copy(ptr_hbm.at[pl.ds(base, _G)], idx_v)

    def mk_gidx(j, _):
        v = idx_v[pl.ds(j * 16, 16)]
        s16 = jnp.zeros((16,), jnp.int32)
        for t in range(1, 8):
            s16 = s16 + jnp.where(v >= t * _SL, 1, 0).astype(jnp.int32)
        gidx_v[pl.ds(j * 16, 16)] = v - s16 * _SL
        olane_v[pl.ds(j * 16, 16)] = s16 * _D
        return 0
    lax.fori_loop(0, _G // 16, mk_gidx, 0)

    def issue(c):
        return pltpu.async_copy(
            e_hbm.at[gidx_v.at[pl.ds(c * _GCH, _GCH)]],
            bufs[c % 3], sems[c % 3])

    pending = [None, None, None]
    pending[0] = issue(0)
    pending[1] = issue(1)
    pending[0].wait()

    def sub16(buf_lo, i_lo, o_lo, buf_hi, i_hi, o_hi):
        lo = buf_lo[i_lo, pl.ds(o_lo, _D)] + off_v[pl.ds(o_lo, _D)]
        hi = buf_hi[i_hi, pl.ds(o_hi, _D)] + off_v[pl.ds(o_hi, _D)]
        return hi - lo

    for c in range(_NG):
        if c + 2 < _NG:
            pending[(c + 2) % 3] = issue(c + 2)
        if c + 1 < _NG:
            pending[(c + 1) % 3].wait()
        ba = bufs[c % 3]
        nrows = min(_GCH, _BPW - c * _GCH)
        in_chunk = nrows if nrows < _GCH else _GCH - 1

        def row(i, _):
            s = c * _GCH + i
            ov = olane_v[pl.ds(s, 16)]
            out_v[s >> 3, pl.ds((s & 7) * _D, _D)] = sub16(
                ba, i, ov[0], ba, i + 1, ov[1])
            return 0
        lax.fori_loop(0, in_chunk, row, 0)
        if in_chunk < nrows:
            s = c * _GCH + _GCH - 1
            ov = olane_v[pl.ds(s, 16)]
            out_v[s >> 3, pl.ds((s & 7) * _D, _D)] = sub16(
                ba, _GCH - 1, ov[0], bufs[(c + 1) % 3], 0, ov[1])

    pltpu.sync_copy(out_v, out_hbm.at[pl.ds(wid * (_BPW // 8), _BPW // 8)])


def kernel(h_node, x, ptr, W1, b1, W2, b2, W3, b3):
    n = h_node.shape[0]
    nf = x.shape[1]
    n8 = n // 8

    xt = x.T
    ht = h_node.T

    w1x = W1[:, :nf]
    w1h = W1[:, nf:]
    w1big = jnp.zeros((256, 192), jnp.float32)
    w2big = jnp.zeros((128, 256), jnp.float32)
    w3big = jnp.zeros((128, 128), jnp.float32)
    for s in range(8):
        w1big = w1big.at[32 * s:32 * s + 32, 24 * s:24 * s + 5].set(w1x)
        w1big = w1big.at[32 * s:32 * s + 32, 24 * s + 8:24 * s + 24].set(w1h)
        w2big = w2big.at[16 * s:16 * s + 16, 32 * s:32 * s + 32].set(W2)
        w3big = w3big.at[16 * s:16 * s + 16, 16 * s:16 * s + 16].set(W3)
    b1big = jnp.tile(b1, 8).reshape(-1, 1)
    b2big = jnp.tile(b2, 8).reshape(-1, 1)
    b3big = jnp.tile(b3, 8).reshape(-1, 1)
    us = jnp.asarray(np.triu(np.ones((_CL, _CL)), 1), dtype=jnp.float32)

    nlastblk = (n + _GL - 1) // _GL - 1

    def _mkmap(s):
        if (s + 1) * _SL <= n:
            return lambda i: (0, s * _NBLK + i)
        return lambda i: (0, jnp.minimum(s * _NBLK + i, nlastblk))

    xspec = [pl.BlockSpec((nf, _GL), _mkmap(s)) for s in range(8)]
    hspec = [pl.BlockSpec((_D, _GL), _mkmap(s)) for s in range(8)]

    e2, tots = pl.pallas_call(
        _mlp_cumsum_body,
        grid=(_NBLK,),
        in_specs=xspec + hspec + [
            pl.BlockSpec(w1big.shape, lambda i: (0, 0)),
            pl.BlockSpec(b1big.shape, lambda i: (0, 0)),
            pl.BlockSpec(w2big.shape, lambda i: (0, 0)),
            pl.BlockSpec(b2big.shape, lambda i: (0, 0)),
            pl.BlockSpec(w3big.shape, lambda i: (0, 0)),
            pl.BlockSpec(b3big.shape, lambda i: (0, 0)),
            pl.BlockSpec(us.shape, lambda i: (0, 0)),
        ],
        out_specs=[pl.BlockSpec((_GL, 8 * _D), lambda i: (i, 0)),
                   pl.BlockSpec((128, 1), lambda i: (0, 0))],
        out_shape=[jax.ShapeDtypeStruct((_SL, 8 * _D), jnp.float32),
                   jax.ShapeDtypeStruct((128, 1), jnp.float32)],
        scratch_shapes=[pltpu.VMEM((128, 1), jnp.float32)],
    )(*([xt] * 8 + [ht] * 8 + [w1big, b1big, w2big, b2big, w3big, b3big, us]))

    ptr32 = jnp.pad(ptr.astype(jnp.int32), (0, _PTR_LEN - (_B + 1)),
                    mode="edge")

    t8 = tots.reshape(8, _D)
    off_flat = (jnp.cumsum(t8, axis=0) - t8).reshape(8 * _D)

    seg = pl.kernel(
        _seg_diff_body,
        out_type=jax.ShapeDtypeStruct((_B_PAD // 8, 8 * _D), jnp.float32),
        mesh=plsc.VectorSubcoreMesh(core_axis_name="c", subcore_axis_name="s"),
        scratch_types=[
            pltpu.VMEM((_G,), jnp.int32),
            pltpu.VMEM((_G,), jnp.int32),
            pltpu.VMEM((_G,), jnp.int32),
            pltpu.VMEM((8 * _D,), jnp.float32),
            pltpu.VMEM((_BPW // 8, 8 * _D), jnp.float32),
            pltpu.VMEM((_GCH, 8 * _D), jnp.float32),
            pltpu.VMEM((_GCH, 8 * _D), jnp.float32),
            pltpu.VMEM((_GCH, 8 * _D), jnp.float32),
            pltpu.SemaphoreType.DMA,
            pltpu.SemaphoreType.DMA,
            pltpu.SemaphoreType.DMA,
        ],
    )(e2, ptr32, off_flat)

    return seg.reshape(_B_PAD, _D)[:_B]
